# baseline (speedup 1.0000x reference)
import functools

import jax
import jax.numpy as jnp
from jax import lax
from jax.experimental import pallas as pl
from jax.experimental.pallas import tpu as pltpu
from jax.experimental.pallas import tpu_sc as plsc

B = 4096
L = 200
D = 64
NOUT = 3

NC = 2
NS = 16
NW = NC * NS
BPW = B // NW

CHUNKS = ((0, 0, 0, 128), (0, 128, 128, 72), (1, 0, 200, 128), (1, 128, 328, 72))
ROWS = 400


def _sc_body(idxh_hbm, idxp_hbm, lenh_hbm, lenp_hbm, embed_hbm, w_hbm,
             bias_hbm, out_hbm, idxh_v, idxp_v, lenh_v, lenp_v, w_v, bias_v,
             rowbuf, outbuf, semh0, semh1, semp0, semp1):
  wid = lax.axis_index("s") * NC + lax.axis_index("c")
  base = wid * BPW

  pltpu.sync_copy(idxh_hbm.at[pl.ds(base, BPW)], idxh_v)
  pltpu.sync_copy(idxp_hbm.at[pl.ds(base, BPW)], idxp_v)
  pltpu.sync_copy(lenh_hbm.at[pl.ds(base, BPW)], lenh_v.at[pl.ds(0, BPW)])
  pltpu.sync_copy(lenp_hbm.at[pl.ds(base, BPW)], lenp_v.at[pl.ds(0, BPW)])
  pltpu.sync_copy(w_hbm, w_v)
  pltpu.sync_copy(bias_hbm, bias_v)

  wv = [[w_v[pl.ds(o * D + k * 16, 16)] for k in range(4)]
        for o in range(NOUT)]
  bias = bias_v[...]
  lane = lax.iota(jnp.int32, 16)
  zero16 = jnp.zeros((16,), jnp.float32)
  sems = ((semh0, semh1), (semp0, semp1))
  idx_refs = (idxh_v, idxp_v)

  def start(ib, s, buf):
    for cs, c0, r0, n in CHUNKS:
      if cs == s:
        pltpu.async_copy(
            embed_hbm.at[idx_refs[s].at[ib, pl.ds(c0, n)]],
            rowbuf.at[buf, pl.ds(r0, n)],
            sems[s][buf])

  def drain(s, buf):
    pltpu.make_async_copy(
        embed_hbm.at[pl.ds(0, L)], rowbuf.at[buf, pl.ds(s * L, L)],
        sems[s][buf]).wait()

  def accumulate(buf, r0, nrows):
    rpi = 20

    def step(t, accs):
      accs = list(accs)
      for j in range(rpi):
        r = r0 + t * rpi + j
        for k in range(4):
          a = (j % 2) * 4 + k
          accs[a] = accs[a] + rowbuf[buf, r, pl.ds(k * 16, 16)]
      return tuple(accs)

    accs = lax.fori_loop(0, nrows // rpi, step, (zero16,) * 8)
    return [accs[k] + accs[4 + k] for k in range(4)]

  start(0, 0, 0)
  start(0, 1, 0)
  start(1, 0, 1)
  start(1, 1, 1)

  @pl.loop(0, BPW, step=2)
  def _(i):
    for buf in range(2):
      ib = i + buf
      drain(0, buf)
      acc_h = accumulate(buf, 0, 200)

      @pl.when(ib + 2 < BPW)
      def _():
        start(ib + 2, 0, buf)

      drain(1, buf)
      acc_p = accumulate(buf, 200, 200)

      @pl.when(ib + 2 < BPW)
      def _():
        start(ib + 2, 1, buf)

      invh = jnp.full((16,), lenh_v[pl.ds(ib, 16)][0])
      invp = jnp.full((16,), lenp_v[pl.ds(ib, 16)][0])
      pooled = [acc_h[k] * invh + acc_p[k] * invp for k in range(4)]

      outv = bias
      for o in range(NOUT):
        prod = (pooled[0] * wv[o][0] + pooled[1] * wv[o][1]
                + pooled[2] * wv[o][2] + pooled[3] * wv[o][3])
        s = jnp.sum(prod)
        outv = outv + jnp.where(lane == o, jnp.full((16,), s), 0.0)
      outbuf[ib, :] = outv

  pltpu.sync_copy(outbuf, out_hbm.at[pl.ds(base, BPW)])


@jax.jit
def kernel(data_hypo, length_hypo, data_prem, length_prem, embed, W, b):
  inv_lenh = 1.0 / length_hypo.astype(jnp.float32)
  inv_lenp = 1.0 / length_prem.astype(jnp.float32)
  w_flat = W.reshape(-1)
  bias_vec = jnp.zeros((16,), jnp.float32).at[:NOUT].set(b)

  run = pl.kernel(
      _sc_body,
      out_type=jax.ShapeDtypeStruct((B, 16), jnp.float32),
      mesh=plsc.VectorSubcoreMesh(
          core_axis_name="c", subcore_axis_name="s",
          num_cores=NC, num_subcores=NS),
      compiler_params=pltpu.CompilerParams(
          needs_layout_passes=False, use_tc_tiling_on_sc=False),
      scratch_types=[
          pltpu.VMEM((BPW, L), jnp.int32),
          pltpu.VMEM((BPW, L), jnp.int32),
          pltpu.VMEM((BPW + 16,), jnp.float32),
          pltpu.VMEM((BPW + 16,), jnp.float32),
          pltpu.VMEM((NOUT * D,), jnp.float32),
          pltpu.VMEM((16,), jnp.float32),
          pltpu.VMEM((2, ROWS, D), jnp.float32),
          pltpu.VMEM((BPW, 16), jnp.float32),
          pltpu.SemaphoreType.DMA,
          pltpu.SemaphoreType.DMA,
          pltpu.SemaphoreType.DMA,
          pltpu.SemaphoreType.DMA,
      ],
  )
  out_pad = run(data_hypo, data_prem, inv_lenh, inv_lenp, embed, w_flat,
                bias_vec)
  return out_pad[:, :NOUT]


# speedup vs baseline: 1.0025x; 1.0025x over previous
import jax
import jax.numpy as jnp
from jax import lax
from jax.experimental import pallas as pl
from jax.experimental.pallas import tpu as pltpu
from jax.experimental.pallas import tpu_sc as plsc

B = 4096
L = 200
D = 64
NOUT = 3

NC = 2
NS = 16
NW = NC * NS
BPW = B // NW

CHUNKS = ((0, 0, 0, 128), (0, 128, 128, 72), (1, 0, 200, 128), (1, 128, 328, 72))
ROWS = 400


def _sc_body(idxh_hbm, idxp_hbm, lenh_hbm, lenp_hbm, embed_hbm, w_hbm,
             bias_hbm, out_hbm, idxh_v, idxp_v, lenh_v, lenp_v, w_v, bias_v,
             rowbuf, outbuf, semh0, semh1, semp0, semp1):
  wid = lax.axis_index("s") * NC + lax.axis_index("c")
  base = wid * BPW

  pltpu.sync_copy(idxh_hbm.at[pl.ds(base, BPW)], idxh_v)
  pltpu.sync_copy(idxp_hbm.at[pl.ds(base, BPW)], idxp_v)
  pltpu.sync_copy(lenh_hbm.at[pl.ds(base, BPW)], lenh_v.at[pl.ds(0, BPW)])
  pltpu.sync_copy(lenp_hbm.at[pl.ds(base, BPW)], lenp_v.at[pl.ds(0, BPW)])
  pltpu.sync_copy(w_hbm, w_v)
  pltpu.sync_copy(bias_hbm, bias_v)

  wv = [[w_v[pl.ds(o * D + k * 16, 16)] for k in range(4)]
        for o in range(NOUT)]
  bias = bias_v[...]
  lane = lax.iota(jnp.int32, 16)
  zero16 = jnp.zeros((16,), jnp.float32)
  sems = ((semh0, semh1), (semp0, semp1))
  idx_refs = (idxh_v, idxp_v)

  def start(ib, s, buf):
    for cs, c0, r0, n in CHUNKS:
      if cs == s:
        pltpu.async_copy(
            embed_hbm.at[idx_refs[s].at[ib, pl.ds(c0, n)]],
            rowbuf.at[buf, pl.ds(r0, n)],
            sems[s][buf])

  def drain(s, buf):
    pltpu.make_async_copy(
        embed_hbm.at[pl.ds(0, L)], rowbuf.at[buf, pl.ds(s * L, L)],
        sems[s][buf]).wait()

  def accumulate(buf, r0, nrows):
    rpi = 8

    def step(t, accs):
      accs = list(accs)
      for j in range(rpi):
        r = r0 + t * rpi + j
        for k in range(4):
          a = (j % 2) * 4 + k
          accs[a] = accs[a] + rowbuf[buf, r, pl.ds(k * 16, 16)]
      return tuple(accs)

    accs = lax.fori_loop(0, nrows // rpi, step, (zero16,) * 8)
    return [accs[k] + accs[4 + k] for k in range(4)]

  start(0, 0, 0)
  start(0, 1, 0)
  start(1, 0, 1)
  start(1, 1, 1)

  @pl.loop(0, BPW, step=2)
  def _(i):
    for buf in range(2):
      ib = i + buf
      drain(0, buf)
      acc_h = accumulate(buf, 0, 200)

      @pl.when(ib + 2 < BPW)
      def _():
        start(ib + 2, 0, buf)

      drain(1, buf)
      acc_p = accumulate(buf, 200, 200)

      @pl.when(ib + 2 < BPW)
      def _():
        start(ib + 2, 1, buf)

      invh = jnp.full((16,), lenh_v[pl.ds(ib, 16)][0])
      invp = jnp.full((16,), lenp_v[pl.ds(ib, 16)][0])
      pooled = [acc_h[k] * invh + acc_p[k] * invp for k in range(4)]

      outv = bias
      for o in range(NOUT):
        prod = (pooled[0] * wv[o][0] + pooled[1] * wv[o][1]
                + pooled[2] * wv[o][2] + pooled[3] * wv[o][3])
        s = jnp.sum(prod)
        outv = outv + jnp.where(lane == o, jnp.full((16,), s), 0.0)
      outbuf[ib, :] = outv

  pltpu.sync_copy(outbuf, out_hbm.at[pl.ds(base, BPW)])


@jax.jit
def kernel(data_hypo, length_hypo, data_prem, length_prem, embed, W, b):
  inv_lenh = 1.0 / length_hypo.astype(jnp.float32)
  inv_lenp = 1.0 / length_prem.astype(jnp.float32)
  w_flat = W.reshape(-1)
  bias_vec = jnp.zeros((16,), jnp.float32).at[:NOUT].set(b)

  run = pl.kernel(
      _sc_body,
      out_type=jax.ShapeDtypeStruct((B, 16), jnp.float32),
      mesh=plsc.VectorSubcoreMesh(
          core_axis_name="c", subcore_axis_name="s",
          num_cores=NC, num_subcores=NS),
      compiler_params=pltpu.CompilerParams(
          needs_layout_passes=False, use_tc_tiling_on_sc=False),
      scratch_types=[
          pltpu.VMEM((BPW, L), jnp.int32),
          pltpu.VMEM((BPW, L), jnp.int32),
          pltpu.VMEM((BPW + 16,), jnp.float32),
          pltpu.VMEM((BPW + 16,), jnp.float32),
          pltpu.VMEM((NOUT * D,), jnp.float32),
          pltpu.VMEM((16,), jnp.float32),
          pltpu.VMEM((2, ROWS, D), jnp.float32),
          pltpu.VMEM((BPW, 16), jnp.float32),
          pltpu.SemaphoreType.DMA,
          pltpu.SemaphoreType.DMA,
          pltpu.SemaphoreType.DMA,
          pltpu.SemaphoreType.DMA,
      ],
  )
  out_pad = run(data_hypo, data_prem, inv_lenh, inv_lenp, embed, w_flat,
                bias_vec)
  return out_pad[:, :NOUT]
